# baseline (speedup 1.0000x reference)
import functools

import jax
import jax.numpy as jnp
from jax import lax
from jax.experimental import pallas as pl
from jax.experimental.pallas import tpu as pltpu
from jax.experimental.pallas import tpu_sc as plsc

N = 10000
E = 320000
D = 128

NC = 2
NS = 16
NW = NC * NS
CHUNK = 32
EPT = E // NW
CPT = ((-(-EPT // CHUNK) + 7) // 8) * 8
PAD_PER_TILE = CPT * CHUNK - EPT
NROWS = NW * CPT
CPD = NROWS // NS
N_PAD = 10240
RPT = N_PAD // NS

_mesh = plsc.VectorSubcoreMesh(core_axis_name="c", subcore_axis_name="s")


DHALF = CPD // 2


@functools.partial(
    pl.kernel,
    out_type=jax.ShapeDtypeStruct((2, N_PAD), jnp.float32),
    mesh=_mesh,
    scratch_types=[
        pltpu.VMEM((DHALF, CHUNK), jnp.int32),
        pltpu.VMEM((CHUNK,), jnp.float32),
        pltpu.VMEM((RPT,), jnp.float32),
        pltpu.VMEM_SHARED((N_PAD,), jnp.float32),
        [pltpu.SemaphoreType.DMA for _ in range(4)],
    ],
)
def _deg_kernel(src_hbm, dst_hbm, out_hbm, idx_v, ones_v, zer_v, acc_sh,
                dsem):
    c = lax.axis_index("c")
    s = lax.axis_index("s")

    def fill_ones(i, _):
        ones_v[pl.ds(i * 16, 16)] = jnp.full((16,), 1.0, jnp.float32)
        return 0

    lax.fori_loop(0, CHUNK // 16, fill_ones, 0)

    def fill_zeros(i, _):
        zer_v[pl.ds(i * 16, 16)] = jnp.zeros((16,), jnp.float32)
        return 0

    lax.fori_loop(0, RPT // 16, fill_zeros, 0)

    pltpu.sync_copy(zer_v, acc_sh.at[pl.ds(s * RPT, RPT)])
    plsc.subcore_barrier()

    def run(edge_hbm):
        def start(j, b):
            pltpu.async_copy(ones_v, acc_sh.at[idx_v.at[j]], dsem[b],
                             add=True)

        def wait(j, b):
            pltpu.make_async_copy(ones_v, acc_sh.at[idx_v.at[j]],
                                  dsem[b]).wait()

        for h in range(2):
            pltpu.sync_copy(edge_hbm.at[pl.ds(s * CPD + h * DHALF, DHALF)],
                            idx_v)

            def quad(m, _):
                for i in range(4):
                    j = 4 * m + i

                    @pl.when(j >= 4)
                    def _():
                        wait(j - 4, i)

                    start(j, i)
                return 0

            lax.fori_loop(0, DHALF // 4, quad, 0)
            for i in range(4):
                wait(DHALF - 4 + i, i)

    @pl.when(c == 0)
    def _():
        run(src_hbm)

    @pl.when(c == 1)
    def _():
        run(dst_hbm)

    plsc.subcore_barrier()
    pltpu.sync_copy(acc_sh.at[pl.ds(s * RPT, RPT)],
                    out_hbm.at[c, pl.ds(s * RPT, RPT)])


ZR = 16
NHALF = 8
CPH = CPT // NHALF
NBUF = 8


@functools.partial(
    pl.kernel,
    out_type=jax.ShapeDtypeStruct((2, N_PAD, D), jnp.float32),
    mesh=_mesh,
    scratch_types=[
        pltpu.VMEM((CPH, CHUNK), jnp.int32),
        pltpu.VMEM((CPH, CHUNK), jnp.int32),
        [pltpu.VMEM((CHUNK, D), jnp.float32) for _ in range(NBUF)],
        pltpu.VMEM((ZR, D), jnp.float32),
        pltpu.VMEM_SHARED((N_PAD, D), jnp.float32),
        [pltpu.SemaphoreType.DMA for _ in range(NBUF)],
        [pltpu.SemaphoreType.DMA for _ in range(NBUF)],
    ],
)
def _edge_kernel(h_hbm, src_hbm, dst_hbm, out_hbm,
                 sidx, didx, bufs, zbuf, acc_sh, gsem, ssem):
    c = lax.axis_index("c")
    s = lax.axis_index("s")
    wid = s * NC + c
    base = wid * CPT

    def zfill(r, _):
        for k in range(D // 16):
            zbuf[r, pl.ds(k * 16, 16)] = jnp.zeros((16,), jnp.float32)
        return 0

    lax.fori_loop(0, ZR, zfill, 0)

    def zero_acc(i, _):
        pltpu.sync_copy(zbuf, acc_sh.at[pl.ds(s * RPT + i * ZR, ZR)])
        return 0

    lax.fori_loop(0, RPT // ZR, zero_acc, 0)
    plsc.subcore_barrier()

    def start_g(j, b):
        pltpu.async_copy(h_hbm.at[sidx.at[j]], bufs[b], gsem[b])

    def wait_g(j, b):
        pltpu.make_async_copy(h_hbm.at[sidx.at[j]], bufs[b], gsem[b]).wait()

    def start_s(j, b):
        pltpu.async_copy(bufs[b], acc_sh.at[didx.at[j]], ssem[b], add=True)

    def wait_s(j, b):
        pltpu.make_async_copy(bufs[b], acc_sh.at[didx.at[j]], ssem[b]).wait()

    for h in range(NHALF):
        pltpu.sync_copy(src_hbm.at[pl.ds(base + h * CPH, CPH)], sidx)
        pltpu.sync_copy(dst_hbm.at[pl.ds(base + h * CPH, CPH)], didx)

        for b in range(NBUF - 1):
            start_g(b, b)

        def ring_body(m, _):
            for i in range(NBUF):
                j = NBUF * m + i
                wait_g(j, i)
                start_s(j, i)
                bn = (i + NBUF - 1) % NBUF

                @pl.when(j >= 1)
                def _():
                    wait_s(j - 1, bn)

                @pl.when(j + NBUF - 1 < CPH)
                def _():
                    start_g(j + NBUF - 1, bn)
            return 0

        lax.fori_loop(0, CPH // NBUF, ring_body, 0)
        wait_s(CPH - 1, (CPH - 1) % NBUF)

    plsc.subcore_barrier()
    pltpu.sync_copy(acc_sh.at[pl.ds(s * RPT, RPT)],
                    out_hbm.at[c, pl.ds(s * RPT, RPT)])


BR = 400
GRID = N // BR


def _norm_body(x_ref, dego_ref, degi_ref, h1_ref, ns_ref, nd_ref):
    dego = dego_ref[...]
    degi = degi_ref[...]
    ns = jnp.where(dego > 0, lax.rsqrt(dego), 0.0)
    nd = jnp.where(degi > 0, lax.rsqrt(degi), 0.0)
    ns_ref[...] = ns
    nd_ref[...] = nd
    h1_ref[...] = x_ref[...] * ns


_norm_call = pl.pallas_call(
    _norm_body,
    grid=(GRID,),
    in_specs=[
        pl.BlockSpec((BR, D), lambda i: (i, 0)),
        pl.BlockSpec((BR, 1), lambda i: (i, 0)),
        pl.BlockSpec((BR, 1), lambda i: (i, 0)),
    ],
    out_specs=[
        pl.BlockSpec((BR, D), lambda i: (i, 0)),
        pl.BlockSpec((BR, 1), lambda i: (i, 0)),
        pl.BlockSpec((BR, 1), lambda i: (i, 0)),
    ],
    out_shape=[
        jax.ShapeDtypeStruct((N, D), jnp.float32),
        jax.ShapeDtypeStruct((N, 1), jnp.float32),
        jax.ShapeDtypeStruct((N, 1), jnp.float32),
    ],
)


def _mm_body_scaled(p_ref, nd_ref, w_ref, b_ref, ns_ref, o_ref):
    p = (p_ref[0] + p_ref[1]) * nd_ref[...]
    y = jnp.dot(p, w_ref[...], preferred_element_type=jnp.float32) + b_ref[...]
    o_ref[...] = y * ns_ref[...]


def _mm_body_plain(p_ref, nd_ref, w_ref, b_ref, o_ref):
    p = (p_ref[0] + p_ref[1]) * nd_ref[...]
    y = jnp.dot(p, w_ref[...], preferred_element_type=jnp.float32) + b_ref[...]
    o_ref[...] = y


def _make_mm(scaled):
    in_specs = [
        pl.BlockSpec((2, BR, D), lambda i: (0, i, 0)),
        pl.BlockSpec((BR, 1), lambda i: (i, 0)),
        pl.BlockSpec((D, D), lambda i: (0, 0)),
        pl.BlockSpec((1, D), lambda i: (0, 0)),
    ]
    if scaled:
        in_specs.append(pl.BlockSpec((BR, 1), lambda i: (i, 0)))
    return pl.pallas_call(
        _mm_body_scaled if scaled else _mm_body_plain,
        grid=(GRID,),
        in_specs=in_specs,
        out_specs=pl.BlockSpec((BR, D), lambda i: (i, 0)),
        out_shape=jax.ShapeDtypeStruct((N, D), jnp.float32),
    )


_mm_scaled = _make_mm(True)
_mm_plain = _make_mm(False)


def kernel(in_feat, edge_index, W1, b1, W2, b2):
    src = edge_index[0]
    dst = edge_index[1]

    pad_i = jnp.arange(PAD_PER_TILE, dtype=jnp.int32)
    gat_pad = jnp.broadcast_to((pad_i * 89) % N, (NW, PAD_PER_TILE))
    dis_pad = jnp.broadcast_to(N + (pad_i % (N_PAD - N)), (NW, PAD_PER_TILE))

    src2 = src.reshape(NW, EPT)
    dst2 = dst.reshape(NW, EPT)
    src_gat = jnp.concatenate([src2, gat_pad], axis=1).reshape(NROWS, CHUNK)
    src_deg = jnp.concatenate([src2, dis_pad], axis=1).reshape(NROWS, CHUNK)
    dst_deg = jnp.concatenate([dst2, dis_pad], axis=1).reshape(NROWS, CHUNK)

    deg = _deg_kernel(src_deg, dst_deg)
    dego = deg[0, :N].reshape(N, 1)
    degi = deg[1, :N].reshape(N, 1)

    h1, ns, nd = _norm_call(in_feat, dego, degi)

    p1 = _edge_kernel(h1, src_gat, dst_deg)
    h2 = _mm_scaled(p1, nd, W1, b1.reshape(1, D), ns)

    p2 = _edge_kernel(h2, src_gat, dst_deg)
    out = _mm_plain(p2, nd, W2, b2.reshape(1, D))
    return out


# speedup vs baseline: 1.0274x; 1.0274x over previous
import functools

import jax
import jax.numpy as jnp
from jax import lax
from jax.experimental import pallas as pl
from jax.experimental.pallas import tpu as pltpu
from jax.experimental.pallas import tpu_sc as plsc

N = 10000
E = 320000
D = 128

NC = 2
NS = 16
NW = NC * NS
CHUNK = 64
EPT = E // NW
CPT = ((-(-EPT // CHUNK) + 7) // 8) * 8
PAD_PER_TILE = CPT * CHUNK - EPT
NROWS = NW * CPT
CPD = NROWS // NS
N_PAD = 10240
RPT = N_PAD // NS

_mesh = plsc.VectorSubcoreMesh(core_axis_name="c", subcore_axis_name="s")


DHALF = CPD // 2


@functools.partial(
    pl.kernel,
    out_type=jax.ShapeDtypeStruct((2, N_PAD), jnp.float32),
    mesh=_mesh,
    scratch_types=[
        pltpu.VMEM((DHALF, CHUNK), jnp.int32),
        pltpu.VMEM((CHUNK,), jnp.float32),
        pltpu.VMEM((RPT,), jnp.float32),
        pltpu.VMEM_SHARED((N_PAD,), jnp.float32),
        [pltpu.SemaphoreType.DMA for _ in range(4)],
    ],
)
def _deg_kernel(src_hbm, dst_hbm, out_hbm, idx_v, ones_v, zer_v, acc_sh,
                dsem):
    c = lax.axis_index("c")
    s = lax.axis_index("s")

    def fill_ones(i, _):
        ones_v[pl.ds(i * 16, 16)] = jnp.full((16,), 1.0, jnp.float32)
        return 0

    lax.fori_loop(0, CHUNK // 16, fill_ones, 0)

    def fill_zeros(i, _):
        zer_v[pl.ds(i * 16, 16)] = jnp.zeros((16,), jnp.float32)
        return 0

    lax.fori_loop(0, RPT // 16, fill_zeros, 0)

    pltpu.sync_copy(zer_v, acc_sh.at[pl.ds(s * RPT, RPT)])
    plsc.subcore_barrier()

    def run(edge_hbm):
        def start(j, b):
            pltpu.async_copy(ones_v, acc_sh.at[idx_v.at[j]], dsem[b],
                             add=True)

        def wait(j, b):
            pltpu.make_async_copy(ones_v, acc_sh.at[idx_v.at[j]],
                                  dsem[b]).wait()

        for h in range(2):
            pltpu.sync_copy(edge_hbm.at[pl.ds(s * CPD + h * DHALF, DHALF)],
                            idx_v)

            def quad(m, _):
                for i in range(4):
                    j = 4 * m + i

                    @pl.when(j >= 4)
                    def _():
                        wait(j - 4, i)

                    start(j, i)
                return 0

            lax.fori_loop(0, DHALF // 4, quad, 0)
            for i in range(4):
                wait(DHALF - 4 + i, i)

    @pl.when(c == 0)
    def _():
        run(src_hbm)

    @pl.when(c == 1)
    def _():
        run(dst_hbm)

    plsc.subcore_barrier()
    pltpu.sync_copy(acc_sh.at[pl.ds(s * RPT, RPT)],
                    out_hbm.at[c, pl.ds(s * RPT, RPT)])


ZR = 16
NHALF = 4
CPH = CPT // NHALF
NBUF = 4


@functools.partial(
    pl.kernel,
    out_type=jax.ShapeDtypeStruct((2, N_PAD, D), jnp.float32),
    mesh=_mesh,
    scratch_types=[
        pltpu.VMEM((CPH, CHUNK), jnp.int32),
        pltpu.VMEM((CPH, CHUNK), jnp.int32),
        [pltpu.VMEM((CHUNK, D), jnp.float32) for _ in range(NBUF)],
        pltpu.VMEM((ZR, D), jnp.float32),
        pltpu.VMEM_SHARED((N_PAD, D), jnp.float32),
        [pltpu.SemaphoreType.DMA for _ in range(NBUF)],
        [pltpu.SemaphoreType.DMA for _ in range(NBUF)],
    ],
)
def _edge_kernel(h_hbm, src_hbm, dst_hbm, out_hbm,
                 sidx, didx, bufs, zbuf, acc_sh, gsem, ssem):
    c = lax.axis_index("c")
    s = lax.axis_index("s")
    wid = s * NC + c
    base = wid * CPT

    def zfill(r, _):
        for k in range(D // 16):
            zbuf[r, pl.ds(k * 16, 16)] = jnp.zeros((16,), jnp.float32)
        return 0

    lax.fori_loop(0, ZR, zfill, 0)

    def zero_acc(i, _):
        pltpu.sync_copy(zbuf, acc_sh.at[pl.ds(s * RPT + i * ZR, ZR)])
        return 0

    lax.fori_loop(0, RPT // ZR, zero_acc, 0)
    plsc.subcore_barrier()

    def start_g(j, b):
        pltpu.async_copy(h_hbm.at[sidx.at[j]], bufs[b], gsem[b])

    def wait_g(j, b):
        pltpu.make_async_copy(h_hbm.at[sidx.at[j]], bufs[b], gsem[b]).wait()

    def start_s(j, b):
        pltpu.async_copy(bufs[b], acc_sh.at[didx.at[j]], ssem[b], add=True)

    def wait_s(j, b):
        pltpu.make_async_copy(bufs[b], acc_sh.at[didx.at[j]], ssem[b]).wait()

    for h in range(NHALF):
        pltpu.sync_copy(src_hbm.at[pl.ds(base + h * CPH, CPH)], sidx)
        pltpu.sync_copy(dst_hbm.at[pl.ds(base + h * CPH, CPH)], didx)

        for b in range(NBUF - 1):
            start_g(b, b)

        def ring_body(m, _):
            for i in range(NBUF):
                j = NBUF * m + i
                wait_g(j, i)
                start_s(j, i)
                bn = (i + NBUF - 1) % NBUF

                @pl.when(j >= 1)
                def _():
                    wait_s(j - 1, bn)

                @pl.when(j + NBUF - 1 < CPH)
                def _():
                    start_g(j + NBUF - 1, bn)
            return 0

        lax.fori_loop(0, CPH // NBUF, ring_body, 0)
        wait_s(CPH - 1, (CPH - 1) % NBUF)

    plsc.subcore_barrier()
    pltpu.sync_copy(acc_sh.at[pl.ds(s * RPT, RPT)],
                    out_hbm.at[c, pl.ds(s * RPT, RPT)])


BR = 400
GRID = N // BR


def _norm_body(x_ref, dego_ref, degi_ref, corr_ref, h1_ref, ns_ref, nd_ref):
    dego = dego_ref[...] - corr_ref[...]
    degi = degi_ref[...]
    ns = jnp.where(dego > 0, lax.rsqrt(dego), 0.0)
    nd = jnp.where(degi > 0, lax.rsqrt(degi), 0.0)
    ns_ref[...] = ns
    nd_ref[...] = nd
    h1_ref[...] = x_ref[...] * ns


_norm_call = pl.pallas_call(
    _norm_body,
    grid=(GRID,),
    in_specs=[
        pl.BlockSpec((BR, D), lambda i: (i, 0)),
        pl.BlockSpec((BR, 1), lambda i: (i, 0)),
        pl.BlockSpec((BR, 1), lambda i: (i, 0)),
        pl.BlockSpec((BR, 1), lambda i: (i, 0)),
    ],
    out_specs=[
        pl.BlockSpec((BR, D), lambda i: (i, 0)),
        pl.BlockSpec((BR, 1), lambda i: (i, 0)),
        pl.BlockSpec((BR, 1), lambda i: (i, 0)),
    ],
    out_shape=[
        jax.ShapeDtypeStruct((N, D), jnp.float32),
        jax.ShapeDtypeStruct((N, 1), jnp.float32),
        jax.ShapeDtypeStruct((N, 1), jnp.float32),
    ],
)


def _zmm_body(x_ref, w_ref, o_ref):
    o_ref[...] = jnp.dot(x_ref[...], w_ref[...],
                         preferred_element_type=jnp.float32)


_zmm_call = pl.pallas_call(
    _zmm_body,
    grid=(GRID,),
    in_specs=[
        pl.BlockSpec((BR, D), lambda i: (i, 0)),
        pl.BlockSpec((D, D), lambda i: (0, 0)),
    ],
    out_specs=pl.BlockSpec((BR, D), lambda i: (i, 0)),
    out_shape=jax.ShapeDtypeStruct((N, D), jnp.float32),
)


def _mid_body(p_ref, nd_ref, b_ref, w_ref, ns_ref, o_ref):
    y = (p_ref[0] + p_ref[1]) * nd_ref[...] + b_ref[...]
    z = jnp.dot(y, w_ref[...], preferred_element_type=jnp.float32)
    o_ref[...] = z * ns_ref[...]


_mid_call = pl.pallas_call(
    _mid_body,
    grid=(GRID,),
    in_specs=[
        pl.BlockSpec((2, BR, D), lambda i: (0, i, 0)),
        pl.BlockSpec((BR, 1), lambda i: (i, 0)),
        pl.BlockSpec((1, D), lambda i: (0, 0)),
        pl.BlockSpec((D, D), lambda i: (0, 0)),
        pl.BlockSpec((BR, 1), lambda i: (i, 0)),
    ],
    out_specs=pl.BlockSpec((BR, D), lambda i: (i, 0)),
    out_shape=jax.ShapeDtypeStruct((N, D), jnp.float32),
)


def _tail_body(p_ref, nd_ref, b_ref, o_ref):
    o_ref[...] = (p_ref[0] + p_ref[1]) * nd_ref[...] + b_ref[...]


_tail_call = pl.pallas_call(
    _tail_body,
    grid=(GRID,),
    in_specs=[
        pl.BlockSpec((2, BR, D), lambda i: (0, i, 0)),
        pl.BlockSpec((BR, 1), lambda i: (i, 0)),
        pl.BlockSpec((1, D), lambda i: (0, 0)),
    ],
    out_specs=pl.BlockSpec((BR, D), lambda i: (i, 0)),
    out_shape=jax.ShapeDtypeStruct((N, D), jnp.float32),
)


import numpy as _np

_CORR = _np.zeros((N, 1), dtype=_np.float32)
_CORR[(_np.arange(PAD_PER_TILE) * 89) % N, 0] += float(NW)


def kernel(in_feat, edge_index, W1, b1, W2, b2):
    src = edge_index[0]
    dst = edge_index[1]

    pad_i = jnp.arange(PAD_PER_TILE, dtype=jnp.int32)
    gat_pad = jnp.broadcast_to((pad_i * 89) % N, (NW, PAD_PER_TILE))
    dis_pad = jnp.broadcast_to(N + (pad_i % (N_PAD - N)), (NW, PAD_PER_TILE))

    src2 = src.reshape(NW, EPT)
    dst2 = dst.reshape(NW, EPT)
    src_gat = jnp.concatenate([src2, gat_pad], axis=1).reshape(NROWS, CHUNK)
    dst_deg = jnp.concatenate([dst2, dis_pad], axis=1).reshape(NROWS, CHUNK)

    deg = _deg_kernel(src_gat, dst_deg)
    z1 = _zmm_call(in_feat, W1)
    dego = deg[0, :N].reshape(N, 1)
    degi = deg[1, :N].reshape(N, 1)

    t1, ns, nd = _norm_call(z1, dego, degi, _CORR)

    p1 = _edge_kernel(t1, src_gat, dst_deg)
    t2 = _mid_call(p1, nd, b1.reshape(1, D), W2, ns)

    p2 = _edge_kernel(t2, src_gat, dst_deg)
    out = _tail_call(p2, nd, b2.reshape(1, D))
    return out


# speedup vs baseline: 1.0368x; 1.0091x over previous
import functools

import jax
import jax.numpy as jnp
from jax import lax
from jax.experimental import pallas as pl
from jax.experimental.pallas import tpu as pltpu
from jax.experimental.pallas import tpu_sc as plsc

N = 10000
E = 320000
D = 128

NC = 2
NS = 16
NW = NC * NS
CHUNK = 64
EPT = E // NW
CPT = ((-(-EPT // CHUNK) + 7) // 8) * 8
PAD_PER_TILE = CPT * CHUNK - EPT
NROWS = NW * CPT
CPD = NROWS // NS
N_PAD = 10240
RPT = N_PAD // NS

_mesh = plsc.VectorSubcoreMesh(core_axis_name="c", subcore_axis_name="s")


DHALF = CPD // 2


@functools.partial(
    pl.kernel,
    out_type=jax.ShapeDtypeStruct((2, N_PAD), jnp.float32),
    mesh=_mesh,
    scratch_types=[
        pltpu.VMEM((DHALF, CHUNK), jnp.int32),
        pltpu.VMEM((CHUNK,), jnp.float32),
        pltpu.VMEM((RPT,), jnp.float32),
        pltpu.VMEM_SHARED((N_PAD,), jnp.float32),
        [pltpu.SemaphoreType.DMA for _ in range(4)],
    ],
)
def _deg_kernel(src_hbm, dst_hbm, out_hbm, idx_v, ones_v, zer_v, acc_sh,
                dsem):
    c = lax.axis_index("c")
    s = lax.axis_index("s")

    def fill_ones(i, _):
        ones_v[pl.ds(i * 16, 16)] = jnp.full((16,), 1.0, jnp.float32)
        return 0

    lax.fori_loop(0, CHUNK // 16, fill_ones, 0)

    def fill_zeros(i, _):
        zer_v[pl.ds(i * 16, 16)] = jnp.zeros((16,), jnp.float32)
        return 0

    lax.fori_loop(0, RPT // 16, fill_zeros, 0)

    pltpu.sync_copy(zer_v, acc_sh.at[pl.ds(s * RPT, RPT)])
    plsc.subcore_barrier()

    def run(edge_hbm):
        def start(j, b):
            pltpu.async_copy(ones_v, acc_sh.at[idx_v.at[j]], dsem[b],
                             add=True)

        def wait(j, b):
            pltpu.make_async_copy(ones_v, acc_sh.at[idx_v.at[j]],
                                  dsem[b]).wait()

        for h in range(2):
            pltpu.sync_copy(edge_hbm.at[pl.ds(s * CPD + h * DHALF, DHALF)],
                            idx_v)

            def quad(m, _):
                for i in range(4):
                    j = 4 * m + i

                    @pl.when(j >= 4)
                    def _():
                        wait(j - 4, i)

                    start(j, i)
                return 0

            lax.fori_loop(0, DHALF // 4, quad, 0)
            for i in range(4):
                wait(DHALF - 4 + i, i)

    @pl.when(c == 0)
    def _():
        run(src_hbm)

    @pl.when(c == 1)
    def _():
        run(dst_hbm)

    plsc.subcore_barrier()
    pltpu.sync_copy(acc_sh.at[pl.ds(s * RPT, RPT)],
                    out_hbm.at[c, pl.ds(s * RPT, RPT)])


ZR = 16
NHALF = 4
CPH = CPT // NHALF
NBUF = 4


@functools.partial(
    pl.kernel,
    out_type=jax.ShapeDtypeStruct((2, N_PAD, D), jnp.float32),
    mesh=_mesh,
    scratch_types=[
        pltpu.VMEM((CPH, CHUNK), jnp.int32),
        pltpu.VMEM((CPH, CHUNK), jnp.int32),
        [pltpu.VMEM((CHUNK, D), jnp.float32) for _ in range(NBUF)],
        pltpu.VMEM((ZR, D), jnp.float32),
        pltpu.VMEM_SHARED((N_PAD, D), jnp.float32),
        [pltpu.SemaphoreType.DMA for _ in range(NBUF)],
        [pltpu.SemaphoreType.DMA for _ in range(NBUF)],
    ],
)
def _edge_kernel(h_hbm, src_hbm, dst_hbm, out_hbm,
                 sidx, didx, bufs, zbuf, acc_sh, gsem, ssem):
    c = lax.axis_index("c")
    s = lax.axis_index("s")
    wid = s * NC + c
    base = wid * CPT

    def zfill(r, _):
        for k in range(D // 16):
            zbuf[r, pl.ds(k * 16, 16)] = jnp.zeros((16,), jnp.float32)
        return 0

    lax.fori_loop(0, ZR, zfill, 0)

    def zero_acc(i, _):
        pltpu.sync_copy(zbuf, acc_sh.at[pl.ds(s * RPT + i * ZR, ZR)])
        return 0

    lax.fori_loop(0, RPT // ZR, zero_acc, 0)
    plsc.subcore_barrier()

    def start_g(j, b):
        pltpu.async_copy(h_hbm.at[sidx.at[j]], bufs[b], gsem[b])

    def wait_g(j, b):
        pltpu.make_async_copy(h_hbm.at[sidx.at[j]], bufs[b], gsem[b]).wait()

    def start_s(j, b):
        pltpu.async_copy(bufs[b], acc_sh.at[didx.at[j]], ssem[b], add=True)

    def wait_s(j, b):
        pltpu.make_async_copy(bufs[b], acc_sh.at[didx.at[j]], ssem[b]).wait()

    for h in range(NHALF):
        pltpu.sync_copy(src_hbm.at[pl.ds(base + h * CPH, CPH)], sidx)
        pltpu.sync_copy(dst_hbm.at[pl.ds(base + h * CPH, CPH)], didx)

        for b in range(NBUF - 1):
            start_g(b, b)

        def ring_body(m, _):
            for i in range(NBUF):
                j = NBUF * m + i
                wait_g(j, i)
                start_s(j, i)
                bn = (i + NBUF - 1) % NBUF

                @pl.when(j >= 1)
                def _():
                    wait_s(j - 1, bn)

                @pl.when(j + NBUF - 1 < CPH)
                def _():
                    start_g(j + NBUF - 1, bn)
            return 0

        lax.fori_loop(0, CPH // NBUF, ring_body, 0)
        wait_s(CPH - 1, (CPH - 1) % NBUF)

    plsc.subcore_barrier()
    pltpu.sync_copy(acc_sh.at[pl.ds(s * RPT, RPT)],
                    out_hbm.at[c, pl.ds(s * RPT, RPT)])


BR = 400
GRID = N // BR


def _norm_body(x_ref, dego_ref, degi_ref, corr_ref, h1_ref, ns_ref, nd_ref):
    dego = dego_ref[...] - corr_ref[...]
    degi = degi_ref[...]
    ns = jnp.where(dego > 0, lax.rsqrt(dego), 0.0)
    nd = jnp.where(degi > 0, lax.rsqrt(degi), 0.0)
    ns_ref[...] = ns
    nd_ref[...] = nd
    h1_ref[...] = x_ref[...] * ns


_norm_call = pl.pallas_call(
    _norm_body,
    grid=(GRID,),
    in_specs=[
        pl.BlockSpec((BR, D), lambda i: (i, 0)),
        pl.BlockSpec((BR, 1), lambda i: (i, 0)),
        pl.BlockSpec((BR, 1), lambda i: (i, 0)),
        pl.BlockSpec((BR, 1), lambda i: (i, 0)),
    ],
    out_specs=[
        pl.BlockSpec((BR, D), lambda i: (i, 0)),
        pl.BlockSpec((BR, 1), lambda i: (i, 0)),
        pl.BlockSpec((BR, 1), lambda i: (i, 0)),
    ],
    out_shape=[
        jax.ShapeDtypeStruct((N, D), jnp.float32),
        jax.ShapeDtypeStruct((N, 1), jnp.float32),
        jax.ShapeDtypeStruct((N, 1), jnp.float32),
    ],
)


def _mm_body_scaled(p_ref, nd_ref, w_ref, b_ref, ns_ref, o_ref):
    p = (p_ref[0] + p_ref[1]) * nd_ref[...]
    y = jnp.dot(p, w_ref[...], preferred_element_type=jnp.float32) + b_ref[...]
    o_ref[...] = y * ns_ref[...]


def _mm_body_plain(p_ref, nd_ref, w_ref, b_ref, o_ref):
    p = (p_ref[0] + p_ref[1]) * nd_ref[...]
    y = jnp.dot(p, w_ref[...], preferred_element_type=jnp.float32) + b_ref[...]
    o_ref[...] = y


def _make_mm(scaled):
    in_specs = [
        pl.BlockSpec((2, BR, D), lambda i: (0, i, 0)),
        pl.BlockSpec((BR, 1), lambda i: (i, 0)),
        pl.BlockSpec((D, D), lambda i: (0, 0)),
        pl.BlockSpec((1, D), lambda i: (0, 0)),
    ]
    if scaled:
        in_specs.append(pl.BlockSpec((BR, 1), lambda i: (i, 0)))
    return pl.pallas_call(
        _mm_body_scaled if scaled else _mm_body_plain,
        grid=(GRID,),
        in_specs=in_specs,
        out_specs=pl.BlockSpec((BR, D), lambda i: (i, 0)),
        out_shape=jax.ShapeDtypeStruct((N, D), jnp.float32),
    )


_mm_scaled = _make_mm(True)
_mm_plain = _make_mm(False)


import numpy as _np

_CORR = _np.zeros((N, 1), dtype=_np.float32)
_CORR[(_np.arange(PAD_PER_TILE) * 89) % N, 0] += float(NW)


def kernel(in_feat, edge_index, W1, b1, W2, b2):
    src = edge_index[0]
    dst = edge_index[1]

    pad_i = jnp.arange(PAD_PER_TILE, dtype=jnp.int32)
    gat_pad = jnp.broadcast_to((pad_i * 89) % N, (NW, PAD_PER_TILE))
    dis_pad = jnp.broadcast_to(N + (pad_i % (N_PAD - N)), (NW, PAD_PER_TILE))

    src2 = src.reshape(NW, EPT)
    dst2 = dst.reshape(NW, EPT)
    src_gat = jnp.concatenate([src2, gat_pad], axis=1).reshape(NROWS, CHUNK)
    dst_deg = jnp.concatenate([dst2, dis_pad], axis=1).reshape(NROWS, CHUNK)

    deg = _deg_kernel(src_gat, dst_deg)
    dego = deg[0, :N].reshape(N, 1)
    degi = deg[1, :N].reshape(N, 1)

    h1, ns, nd = _norm_call(in_feat, dego, degi, _CORR)

    p1 = _edge_kernel(h1, src_gat, dst_deg)
    h2 = _mm_scaled(p1, nd, W1, b1.reshape(1, D), ns)

    p2 = _edge_kernel(h2, src_gat, dst_deg)
    out = _mm_plain(p2, nd, W2, b2.reshape(1, D))
    return out


# speedup vs baseline: 1.0588x; 1.0212x over previous
import functools

import jax
import jax.numpy as jnp
from jax import lax
from jax.experimental import pallas as pl
from jax.experimental.pallas import tpu as pltpu
from jax.experimental.pallas import tpu_sc as plsc

N = 10000
E = 320000
D = 128

NC = 2
NS = 16
NW = NC * NS
CHUNK = 64
EPT = E // NW
CPT = ((-(-EPT // CHUNK) + 7) // 8) * 8
PAD_PER_TILE = CPT * CHUNK - EPT
NROWS = NW * CPT
CPD = NROWS // NS
N_PAD = 10240
RPT = N_PAD // NS

_mesh = plsc.VectorSubcoreMesh(core_axis_name="c", subcore_axis_name="s")


DHALF = CPD // 2


@functools.partial(
    pl.kernel,
    out_type=jax.ShapeDtypeStruct((2, N_PAD), jnp.float32),
    mesh=_mesh,
    scratch_types=[
        pltpu.VMEM((DHALF, CHUNK), jnp.int32),
        pltpu.VMEM((CHUNK,), jnp.float32),
        pltpu.VMEM((RPT,), jnp.float32),
        pltpu.VMEM_SHARED((N_PAD,), jnp.float32),
        [pltpu.SemaphoreType.DMA for _ in range(4)],
    ],
)
def _deg_kernel(src_hbm, dst_hbm, out_hbm, idx_v, ones_v, zer_v, acc_sh,
                dsem):
    c = lax.axis_index("c")
    s = lax.axis_index("s")

    def fill_ones(i, _):
        ones_v[pl.ds(i * 16, 16)] = jnp.full((16,), 1.0, jnp.float32)
        return 0

    lax.fori_loop(0, CHUNK // 16, fill_ones, 0)

    def fill_zeros(i, _):
        zer_v[pl.ds(i * 16, 16)] = jnp.zeros((16,), jnp.float32)
        return 0

    lax.fori_loop(0, RPT // 16, fill_zeros, 0)

    pltpu.sync_copy(zer_v, acc_sh.at[pl.ds(s * RPT, RPT)])
    plsc.subcore_barrier()

    def run(edge_hbm):
        def start(j, b):
            pltpu.async_copy(ones_v, acc_sh.at[idx_v.at[j]], dsem[b],
                             add=True)

        def wait(j, b):
            pltpu.make_async_copy(ones_v, acc_sh.at[idx_v.at[j]],
                                  dsem[b]).wait()

        for h in range(2):
            pltpu.sync_copy(edge_hbm.at[pl.ds(s * CPD + h * DHALF, DHALF)],
                            idx_v)

            def quad(m, _):
                for i in range(4):
                    j = 4 * m + i

                    @pl.when(j >= 4)
                    def _():
                        wait(j - 4, i)

                    start(j, i)
                return 0

            lax.fori_loop(0, DHALF // 4, quad, 0)
            for i in range(4):
                wait(DHALF - 4 + i, i)

    @pl.when(c == 0)
    def _():
        run(src_hbm)

    @pl.when(c == 1)
    def _():
        run(dst_hbm)

    plsc.subcore_barrier()
    pltpu.sync_copy(acc_sh.at[pl.ds(s * RPT, RPT)],
                    out_hbm.at[c, pl.ds(s * RPT, RPT)])


ZR = 16
NHALF = 4
CPH = CPT // NHALF
NBUF = 4


@functools.partial(
    pl.kernel,
    out_type=jax.ShapeDtypeStruct((2, N_PAD, D), jnp.float32),
    mesh=_mesh,
    scratch_types=[
        pltpu.VMEM((CPH, CHUNK), jnp.int32),
        pltpu.VMEM((CPH, CHUNK), jnp.int32),
        [pltpu.VMEM((CHUNK, D), jnp.float32) for _ in range(NBUF)],
        pltpu.VMEM((ZR, D), jnp.float32),
        pltpu.VMEM_SHARED((N_PAD, D), jnp.float32),
        [pltpu.SemaphoreType.DMA for _ in range(NBUF)],
        [pltpu.SemaphoreType.DMA for _ in range(NBUF)],
    ],
)
def _edge_kernel(h_hbm, src_hbm, dst_hbm, out_hbm,
                 sidx, didx, bufs, zbuf, acc_sh, gsem, ssem):
    c = lax.axis_index("c")
    s = lax.axis_index("s")
    wid = s * NC + c
    base = wid * CPT

    def zfill(r, _):
        for k in range(D // 16):
            zbuf[r, pl.ds(k * 16, 16)] = jnp.zeros((16,), jnp.float32)
        return 0

    lax.fori_loop(0, ZR, zfill, 0)

    pltpu.async_copy(src_hbm.at[pl.ds(base, CPH)], sidx, ssem[0])
    pltpu.async_copy(dst_hbm.at[pl.ds(base, CPH)], didx, ssem[1])

    def zstart(i, b):
        pltpu.async_copy(zbuf, acc_sh.at[pl.ds(s * RPT + i * ZR, ZR)],
                         gsem[b])

    def zwait(i, b):
        pltpu.make_async_copy(zbuf, acc_sh.at[pl.ds(s * RPT + i * ZR, ZR)],
                              gsem[b]).wait()

    def zero_acc(m, _):
        for b in range(4):
            i = 4 * m + b

            @pl.when(i >= 4)
            def _():
                zwait(i - 4, b)

            zstart(i, b)
        return 0

    lax.fori_loop(0, RPT // ZR // 4, zero_acc, 0)
    for b in range(4):
        zwait(RPT // ZR - 4 + b, b)
    pltpu.make_async_copy(src_hbm.at[pl.ds(base, CPH)], sidx, ssem[0]).wait()
    pltpu.make_async_copy(dst_hbm.at[pl.ds(base, CPH)], didx, ssem[1]).wait()
    plsc.subcore_barrier()

    def start_g(j, b):
        pltpu.async_copy(h_hbm.at[sidx.at[j]], bufs[b], gsem[b])

    def wait_g(j, b):
        pltpu.make_async_copy(h_hbm.at[sidx.at[j]], bufs[b], gsem[b]).wait()

    def start_s(j, b):
        pltpu.async_copy(bufs[b], acc_sh.at[didx.at[j]], ssem[b], add=True)

    def wait_s(j, b):
        pltpu.make_async_copy(bufs[b], acc_sh.at[didx.at[j]], ssem[b]).wait()

    for h in range(NHALF):
        if h > 0:
            pltpu.sync_copy(src_hbm.at[pl.ds(base + h * CPH, CPH)], sidx)
            pltpu.sync_copy(dst_hbm.at[pl.ds(base + h * CPH, CPH)], didx)

        for b in range(NBUF - 1):
            start_g(b, b)

        def ring_body(m, _):
            for i in range(NBUF):
                j = NBUF * m + i
                wait_g(j, i)
                start_s(j, i)
                bn = (i + NBUF - 1) % NBUF

                @pl.when(j >= 1)
                def _():
                    wait_s(j - 1, bn)

                @pl.when(j + NBUF - 1 < CPH)
                def _():
                    start_g(j + NBUF - 1, bn)
            return 0

        lax.fori_loop(0, CPH // NBUF, ring_body, 0)
        wait_s(CPH - 1, (CPH - 1) % NBUF)

    plsc.subcore_barrier()
    pltpu.sync_copy(acc_sh.at[pl.ds(s * RPT, RPT)],
                    out_hbm.at[c, pl.ds(s * RPT, RPT)])


BR = 400
GRID = N // BR


def _norm_body(x_ref, dego_ref, degi_ref, corr_ref, h1_ref, ns_ref, nd_ref):
    dego = dego_ref[...] - corr_ref[...]
    degi = degi_ref[...]
    ns = jnp.where(dego > 0, lax.rsqrt(dego), 0.0)
    nd = jnp.where(degi > 0, lax.rsqrt(degi), 0.0)
    ns_ref[...] = ns
    nd_ref[...] = nd
    h1_ref[...] = x_ref[...] * ns


_norm_call = pl.pallas_call(
    _norm_body,
    grid=(GRID,),
    in_specs=[
        pl.BlockSpec((BR, D), lambda i: (i, 0)),
        pl.BlockSpec((BR, 1), lambda i: (i, 0)),
        pl.BlockSpec((BR, 1), lambda i: (i, 0)),
        pl.BlockSpec((BR, 1), lambda i: (i, 0)),
    ],
    out_specs=[
        pl.BlockSpec((BR, D), lambda i: (i, 0)),
        pl.BlockSpec((BR, 1), lambda i: (i, 0)),
        pl.BlockSpec((BR, 1), lambda i: (i, 0)),
    ],
    out_shape=[
        jax.ShapeDtypeStruct((N, D), jnp.float32),
        jax.ShapeDtypeStruct((N, 1), jnp.float32),
        jax.ShapeDtypeStruct((N, 1), jnp.float32),
    ],
)


def _mm_body_scaled(p_ref, nd_ref, w_ref, b_ref, ns_ref, o_ref):
    p = (p_ref[0] + p_ref[1]) * nd_ref[...]
    y = jnp.dot(p, w_ref[...], preferred_element_type=jnp.float32) + b_ref[...]
    o_ref[...] = y * ns_ref[...]


def _mm_body_plain(p_ref, nd_ref, w_ref, b_ref, o_ref):
    p = (p_ref[0] + p_ref[1]) * nd_ref[...]
    y = jnp.dot(p, w_ref[...], preferred_element_type=jnp.float32) + b_ref[...]
    o_ref[...] = y


def _make_mm(scaled):
    in_specs = [
        pl.BlockSpec((2, BR, D), lambda i: (0, i, 0)),
        pl.BlockSpec((BR, 1), lambda i: (i, 0)),
        pl.BlockSpec((D, D), lambda i: (0, 0)),
        pl.BlockSpec((1, D), lambda i: (0, 0)),
    ]
    if scaled:
        in_specs.append(pl.BlockSpec((BR, 1), lambda i: (i, 0)))
    return pl.pallas_call(
        _mm_body_scaled if scaled else _mm_body_plain,
        grid=(GRID,),
        in_specs=in_specs,
        out_specs=pl.BlockSpec((BR, D), lambda i: (i, 0)),
        out_shape=jax.ShapeDtypeStruct((N, D), jnp.float32),
    )


_mm_scaled = _make_mm(True)
_mm_plain = _make_mm(False)


import numpy as _np

_CORR = _np.zeros((N, 1), dtype=_np.float32)
_CORR[(_np.arange(PAD_PER_TILE) * 89) % N, 0] += float(NW)


def kernel(in_feat, edge_index, W1, b1, W2, b2):
    src = edge_index[0]
    dst = edge_index[1]

    pad_i = jnp.arange(PAD_PER_TILE, dtype=jnp.int32)
    gat_pad = jnp.broadcast_to((pad_i * 89) % N, (NW, PAD_PER_TILE))
    dis_pad = jnp.broadcast_to(N + (pad_i % (N_PAD - N)), (NW, PAD_PER_TILE))

    src2 = src.reshape(NW, EPT)
    dst2 = dst.reshape(NW, EPT)
    src_gat = jnp.concatenate([src2, gat_pad], axis=1).reshape(NROWS, CHUNK)
    dst_deg = jnp.concatenate([dst2, dis_pad], axis=1).reshape(NROWS, CHUNK)

    deg = _deg_kernel(src_gat, dst_deg)
    dego = deg[0, :N].reshape(N, 1)
    degi = deg[1, :N].reshape(N, 1)

    h1, ns, nd = _norm_call(in_feat, dego, degi, _CORR)

    p1 = _edge_kernel(h1, src_gat, dst_deg)
    h2 = _mm_scaled(p1, nd, W1, b1.reshape(1, D), ns)

    p2 = _edge_kernel(h2, src_gat, dst_deg)
    out = _mm_plain(p2, nd, W2, b2.reshape(1, D))
    return out
